# baseline (speedup 1.0000x reference)
import jax
import jax.numpy as jnp
from jax import lax
from jax.experimental import pallas as pl
from jax.experimental.pallas import tpu as pltpu
from jax.experimental.pallas import tpu_sc as plsc

NC, NS, L = 2, 16, 16
B = 4096
NT = 26
HT = NT // NC
D = 128
W = D // 2
NG = W // L
NROWS = 26000
HALF = NROWS // NC
RPT = B // NS
PCH = 102
BC = 16
GCH = RPT // BC
GROWS = HT * BC
NIDX = RPT * HT

_mesh = plsc.VectorSubcoreMesh(core_axis_name="c", subcore_axis_name="s")

_scratch = [
    pltpu.VMEM_SHARED((HALF, W), jnp.int32),
    pltpu.VMEM((PCH, D), jnp.float32),
    pltpu.VMEM((PCH, D), jnp.float32),
    pltpu.VMEM((PCH, W), jnp.int32),
    pltpu.VMEM((RPT * NT,), jnp.int32),
    pltpu.VMEM((NIDX + 8,), jnp.int32),
    pltpu.VMEM((GROWS, W), jnp.int32),
    pltpu.VMEM((GROWS, W), jnp.int32),
    pltpu.VMEM((BC, D), jnp.float32),
    pltpu.SemaphoreType.DMA,
    pltpu.SemaphoreType.DMA,
    pltpu.SemaphoreType.DMA,
    pltpu.SemaphoreType.DMA,
]


def _embed_pool_body(act_hbm, table_hbm, out_hbm,
                     spt, fb0, fb1, pbuf, av, idx_v, gb0, gb1, obuf,
                     fsem0, fsem1, gsem0, gsem1):
    k = lax.axis_index("c")
    tid = lax.axis_index("s")
    lanes = lax.iota(jnp.int32, L)

    r0 = tid * 812 + jnp.minimum(tid, 8)
    r1 = (tid + 1) * 812 + jnp.minimum(tid + 1, 8)
    starts = tuple(r0 + c * PCH for c in range(7)) + (r1 - PCH,)
    fbufs = ((fb0, fsem0), (fb1, fsem1))

    def start_pack(c, fb, sem):
        pltpu.async_copy(table_hbm.at[pl.ds(k * HALF + starts[c], PCH)], fb, sem)

    start_pack(0, fb0, fsem0)
    start_pack(1, fb1, fsem1)

    pltpu.sync_copy(act_hbm.at[pl.ds(tid * RPT * NT, RPT * NT)], av)
    offv = lanes * 1000

    @pl.loop(0, RPT)
    def _mkidx(j):
        idx_v[pl.ds(j * HT, L)] = av[pl.ds(j * NT + HT * k, L)] + offv

    def rne(u):
        return lax.shift_right_logical(
            u + 0x7FFF + (lax.shift_right_logical(u, 16) & 1), 16
        )

    for c in range(8):
        fb, sem = fbufs[c % 2]
        pltpu.make_async_copy(
            table_hbm.at[pl.ds(k * HALF + starts[c], PCH)], fb, sem
        ).wait()

        @pl.loop(0, PCH, unroll=2)
        def _pack(r):
            for g in range(NG):
                a = lax.bitcast_convert_type(fb[r, pl.ds(g * 2 * L, L)], jnp.int32)
                b = lax.bitcast_convert_type(fb[r, pl.ds(g * 2 * L + L, L)], jnp.int32)
                pbuf[r, pl.ds(g * L, L)] = rne(a) | (rne(b) << 16)

        pltpu.sync_copy(pbuf, spt.at[pl.ds(starts[c], PCH)])
        if c + 2 < 8:
            start_pack(c + 2, fb, sem)

    plsc.subcore_barrier()

    gbufs = ((gb0, gsem0), (gb1, gsem1))
    zeros = jnp.zeros((L,), jnp.float32)

    def start_gather(c, gb, sem):
        pltpu.async_copy(spt.at[idx_v.at[pl.ds(c * GROWS, GROWS)]], gb, sem)

    start_gather(0, gb0, gsem0)
    start_gather(1, gb1, gsem1)

    obase = k * B + tid * RPT

    @pl.loop(0, GCH, step=2)
    def _chunks(c0):
        for bsel in range(2):
            gb, sem = gbufs[bsel]
            c = c0 + bsel
            pltpu.make_async_copy(
                spt.at[idx_v.at[pl.ds(c * GROWS, GROWS)]], gb, sem
            ).wait()
            for jj in range(BC):
                def body(t, accs):
                    out = []
                    for g in range(NG):
                        w = gb[jj * HT + t, pl.ds(g * L, L)]
                        lo = lax.bitcast_convert_type(w << 16, jnp.float32)
                        hi = lax.bitcast_convert_type((w >> 16) << 16, jnp.float32)
                        out.append(accs[2 * g] + lo)
                        out.append(accs[2 * g + 1] + hi)
                    return tuple(out)

                accs = lax.fori_loop(0, HT, body, (zeros,) * (2 * NG), unroll=2)
                for g in range(NG):
                    obuf[jj, pl.ds(2 * g * L, L)] = accs[2 * g]
                    obuf[jj, pl.ds((2 * g + 1) * L, L)] = accs[2 * g + 1]

            @pl.when(c + 2 < GCH)
            def _():
                start_gather(c + 2, gb, sem)

            pltpu.sync_copy(obuf, out_hbm.at[pl.ds(obase + c * BC, BC)])


_embed_pool = pl.kernel(
    _embed_pool_body,
    out_type=jax.ShapeDtypeStruct((NC * B, D), jnp.float32),
    mesh=_mesh,
    scratch_types=_scratch,
    compiler_params=pltpu.CompilerParams(use_tc_tiling_on_sc=False),
)


def kernel(actions, embed_table):
    act_flat = actions.astype(jnp.int32).reshape(B * NT)
    partial = _embed_pool(act_flat, embed_table)
    return partial[:B] + partial[B:]


# speedup vs baseline: 1.2988x; 1.2988x over previous
import jax
import jax.numpy as jnp
from jax import lax
from jax.experimental import pallas as pl
from jax.experimental.pallas import tpu as pltpu
from jax.experimental.pallas import tpu_sc as plsc

NC, NS, L = 2, 16, 16
NW = NC * NS
B = 4096
NT = 26
D = 128
NV = D // L
BPW = B // NW
BC = 8
NCHUNK = BPW // BC
ROWS = NT * BC
NBUF = 4
NIDX = NT * BPW
PER = 208

_mesh = plsc.VectorSubcoreMesh(core_axis_name="c", subcore_axis_name="s")

_scratch = [
    pltpu.VMEM((NIDX,), jnp.int32),
    pltpu.VMEM((NIDX,), jnp.int32),
    pltpu.VMEM((PER,), jnp.int32),
    pltpu.VMEM((ROWS, D), jnp.float32),
    pltpu.VMEM((ROWS, D), jnp.float32),
    pltpu.VMEM((ROWS, D), jnp.float32),
    pltpu.VMEM((ROWS, D), jnp.float32),
    pltpu.VMEM((BC, D), jnp.float32),
    pltpu.SemaphoreType.DMA,
    pltpu.SemaphoreType.DMA,
    pltpu.SemaphoreType.DMA,
    pltpu.SemaphoreType.DMA,
]


def _embed_pool_body(act_hbm, table_hbm, out_hbm,
                     act_v, idx_v, off_v, gbuf0, gbuf1, gbuf2, gbuf3, obuf,
                     sem0, sem1, sem2, sem3):
    wid = lax.axis_index("s") * NC + lax.axis_index("c")
    base = wid * BPW

    pltpu.sync_copy(act_hbm.at[pl.ds(base * NT, NIDX)], act_v)

    lanes = lax.iota(jnp.int32, L)
    for k in range(0, PER, L):
        off_v[pl.ds(k, L)] = lax.rem(lanes + k, NT) * 1000

    for k in range(0, NIDX, L):
        idx_v[pl.ds(k, L)] = act_v[pl.ds(k, L)] + off_v[pl.ds(k % PER, L)]

    bufs = ((gbuf0, sem0), (gbuf1, sem1), (gbuf2, sem2), (gbuf3, sem3))

    def start_gather(c, buf, sem):
        pltpu.async_copy(table_hbm.at[idx_v.at[pl.ds(c * ROWS, ROWS)]], buf, sem)

    for b in range(NBUF):
        start_gather(b, *bufs[b])

    @pl.loop(0, NCHUNK, step=NBUF)
    def _pair(c0):
        for b in range(NBUF):
            gbuf, sem = bufs[b]
            c = c0 + b
            pltpu.make_async_copy(
                table_hbm.at[idx_v.at[pl.ds(c * ROWS, ROWS)]], gbuf, sem
            ).wait()
            for jj in range(BC):
                def body(t, accs):
                    return tuple(
                        a + gbuf[jj * NT + t, pl.ds(v * L, L)]
                        for v, a in enumerate(accs)
                    )
                accs = tuple(gbuf[jj * NT, pl.ds(v * L, L)] for v in range(NV))
                accs = lax.fori_loop(1, NT, body, accs, unroll=5)
                for v in range(NV):
                    obuf[jj, pl.ds(v * L, L)] = accs[v]

            @pl.when(c + NBUF < NCHUNK)
            def _():
                start_gather(c + NBUF, gbuf, sem)

            pltpu.sync_copy(obuf, out_hbm.at[pl.ds(base + c * BC, BC)])


_embed_pool = pl.kernel(
    _embed_pool_body,
    out_type=jax.ShapeDtypeStruct((B, D), jnp.float32),
    mesh=_mesh,
    scratch_types=_scratch,
)


def kernel(actions, embed_table):
    act_flat = actions.astype(jnp.int32).reshape(B * NT)
    return _embed_pool(act_flat, embed_table)


# speedup vs baseline: 1.3056x; 1.0052x over previous
import jax
import jax.numpy as jnp
from jax import lax
from jax.experimental import pallas as pl
from jax.experimental.pallas import tpu as pltpu
from jax.experimental.pallas import tpu_sc as plsc

NC, NS, L = 2, 16, 16
NW = NC * NS
B = 4096
NT = 26
D = 128
NV = D // L
BPW = B // NW
BC = 8
NCHUNK = BPW // BC
ROWS = NT * BC
NBUF = 4
NIDX = NT * BPW
PER = 208

_mesh = plsc.VectorSubcoreMesh(core_axis_name="c", subcore_axis_name="s")

_scratch = [
    pltpu.VMEM((NIDX,), jnp.int32),
    pltpu.VMEM((NIDX,), jnp.int32),
    pltpu.VMEM((PER,), jnp.int32),
    pltpu.VMEM((ROWS, D), jnp.float32),
    pltpu.VMEM((ROWS, D), jnp.float32),
    pltpu.VMEM((ROWS, D), jnp.float32),
    pltpu.VMEM((ROWS, D), jnp.float32),
    pltpu.VMEM((BC, D), jnp.float32),
    pltpu.VMEM((BC, D), jnp.float32),
    pltpu.SemaphoreType.DMA,
    pltpu.SemaphoreType.DMA,
    pltpu.SemaphoreType.DMA,
    pltpu.SemaphoreType.DMA,
    pltpu.SemaphoreType.DMA,
]


def _embed_pool_body(act_hbm, table_hbm, out_hbm,
                     act_v, idx_v, off_v, gbuf0, gbuf1, gbuf2, gbuf3,
                     obuf0, obuf1, sem0, sem1, sem2, sem3, osem):
    wid = lax.axis_index("s") * NC + lax.axis_index("c")
    base = wid * BPW

    pltpu.sync_copy(act_hbm.at[pl.ds(base * NT, NIDX)], act_v)

    lanes = lax.iota(jnp.int32, L)
    for k in range(0, PER, L):
        off_v[pl.ds(k, L)] = lax.rem(lanes + k, NT) * 1000

    for k in range(0, NIDX, L):
        idx_v[pl.ds(k, L)] = act_v[pl.ds(k, L)] + off_v[pl.ds(k % PER, L)]

    bufs = ((gbuf0, sem0), (gbuf1, sem1), (gbuf2, sem2), (gbuf3, sem3))

    def start_gather(c, buf, sem):
        pltpu.async_copy(table_hbm.at[idx_v.at[pl.ds(c * ROWS, ROWS)]], buf, sem)

    for b in range(NBUF):
        start_gather(b, *bufs[b])

    obufs = (obuf0, obuf1)

    @pl.loop(0, NCHUNK, step=NBUF)
    def _pair(c0):
        for b in range(NBUF):
            gbuf, sem = bufs[b]
            obuf = obufs[b % 2]
            c = c0 + b
            pltpu.make_async_copy(
                table_hbm.at[idx_v.at[pl.ds(c * ROWS, ROWS)]], gbuf, sem
            ).wait()

            @pl.when(c >= 2)
            def _():
                pltpu.make_async_copy(
                    obuf, out_hbm.at[pl.ds(base, BC)], osem
                ).wait()

            for jj in range(BC):
                def body(t, accs):
                    return tuple(
                        a + gbuf[jj * NT + t, pl.ds(v * L, L)]
                        for v, a in enumerate(accs)
                    )
                accs = tuple(gbuf[jj * NT, pl.ds(v * L, L)] for v in range(NV))
                accs = lax.fori_loop(1, NT, body, accs, unroll=5)
                for v in range(NV):
                    obuf[jj, pl.ds(v * L, L)] = accs[v]

            @pl.when(c + NBUF < NCHUNK)
            def _():
                start_gather(c + NBUF, gbuf, sem)

            pltpu.async_copy(obuf, out_hbm.at[pl.ds(base + c * BC, BC)], osem)

    for _ in range(2):
        pltpu.make_async_copy(obuf0, out_hbm.at[pl.ds(base, BC)], osem).wait()


_embed_pool = pl.kernel(
    _embed_pool_body,
    out_type=jax.ShapeDtypeStruct((B, D), jnp.float32),
    mesh=_mesh,
    scratch_types=_scratch,
)


def kernel(actions, embed_table):
    act_flat = actions.astype(jnp.int32).reshape(B * NT)
    return _embed_pool(act_flat, embed_table)


# speedup vs baseline: 1.3298x; 1.0185x over previous
import jax
import jax.numpy as jnp
from jax import lax
from jax.experimental import pallas as pl
from jax.experimental.pallas import tpu as pltpu
from jax.experimental.pallas import tpu_sc as plsc

NC, NS, L = 2, 16, 16
NW = NC * NS
B = 4096
NT = 26
D = 128
NV = D // L
BPW = B // NW
BC = 8
NCHUNK = BPW // BC
ROWS = NT * BC
NBUF = 4
NIDX = NT * BPW
PER = 208

_mesh = plsc.VectorSubcoreMesh(core_axis_name="c", subcore_axis_name="s")

_scratch = [
    pltpu.VMEM((BPW, NT), jnp.int32),
    pltpu.VMEM((NIDX,), jnp.int32),
    pltpu.VMEM((ROWS, D), jnp.float32),
    pltpu.VMEM((ROWS, D), jnp.float32),
    pltpu.VMEM((ROWS, D), jnp.float32),
    pltpu.VMEM((ROWS, D), jnp.float32),
    pltpu.VMEM((BC, D), jnp.float32),
    pltpu.VMEM((BC, D), jnp.float32),
    pltpu.SemaphoreType.DMA,
    pltpu.SemaphoreType.DMA,
    pltpu.SemaphoreType.DMA,
    pltpu.SemaphoreType.DMA,
    pltpu.SemaphoreType.DMA,
]


def _embed_pool_body(act_hbm, table_hbm, out_hbm,
                     act_v, idx_v, gbuf0, gbuf1, gbuf2, gbuf3,
                     obuf0, obuf1, sem0, sem1, sem2, sem3, osem):
    wid = lax.axis_index("s") * NC + lax.axis_index("c")
    base = wid * BPW

    pltpu.sync_copy(act_hbm.at[pl.ds(base, BPW), :], act_v)

    lanes = lax.iota(jnp.int32, L)
    off_a = lanes * 1000
    off_b = off_a + 10000

    @pl.loop(0, BPW)
    def _mkidx(j):
        idx_v[pl.ds(j * NT, L)] = act_v[j, pl.ds(0, L)] + off_a
        idx_v[pl.ds(j * NT + NT - L, L)] = act_v[j, pl.ds(NT - L, L)] + off_b

    bufs = ((gbuf0, sem0), (gbuf1, sem1), (gbuf2, sem2), (gbuf3, sem3))

    def start_gather(c, buf, sem):
        pltpu.async_copy(table_hbm.at[idx_v.at[pl.ds(c * ROWS, ROWS)]], buf, sem)

    for b in range(NBUF):
        start_gather(b, *bufs[b])

    obufs = (obuf0, obuf1)

    @pl.loop(0, NCHUNK, step=NBUF)
    def _pair(c0):
        for b in range(NBUF):
            gbuf, sem = bufs[b]
            obuf = obufs[b % 2]
            c = c0 + b
            pltpu.make_async_copy(
                table_hbm.at[idx_v.at[pl.ds(c * ROWS, ROWS)]], gbuf, sem
            ).wait()

            @pl.when(c >= 2)
            def _():
                pltpu.make_async_copy(
                    obuf, out_hbm.at[pl.ds(base, BC)], osem
                ).wait()

            for jj in range(BC):
                def body(t, accs):
                    return tuple(
                        a + gbuf[jj * NT + t, pl.ds(v * L, L)]
                        for v, a in enumerate(accs)
                    )
                accs = tuple(gbuf[jj * NT, pl.ds(v * L, L)] for v in range(NV))
                accs = lax.fori_loop(1, NT, body, accs, unroll=5)
                for v in range(NV):
                    obuf[jj, pl.ds(v * L, L)] = accs[v]

            @pl.when(c + NBUF < NCHUNK)
            def _():
                start_gather(c + NBUF, gbuf, sem)

            pltpu.async_copy(obuf, out_hbm.at[pl.ds(base + c * BC, BC)], osem)

    for _ in range(2):
        pltpu.make_async_copy(obuf0, out_hbm.at[pl.ds(base, BC)], osem).wait()


_embed_pool = pl.kernel(
    _embed_pool_body,
    out_type=jax.ShapeDtypeStruct((B, D), jnp.float32),
    mesh=_mesh,
    scratch_types=_scratch,
)


def kernel(actions, embed_table):
    return _embed_pool(actions.astype(jnp.int32), embed_table)
